# Initial kernel scaffold; baseline (speedup 1.0000x reference)
import jax
import jax.numpy as jnp
from jax.experimental import pallas as pl


def kernel(inputs):
    raise NotImplementedError("write your pallas kernel here")


# speedup vs baseline: 2.2123x; 2.2123x over previous
import jax
import jax.numpy as jnp
from jax import lax
from jax.experimental import pallas as pl
from jax.experimental.pallas import tpu as pltpu
from jax.experimental.pallas import tpu_sc as plsc

B = 64
HW = 384 * 384
WPI = HW * 3
NBINS = 256
H3 = 3 * NBINS
NC, NS, L = 2, 16, 16
NW = NC * NS
IPW = B // NW
CHUNK = 12288
NCHUNK = WPI // CHUNK


def _body(in_hbm, out_hbm, buf0, buf1, hist, out_buf, sem0, sem1):
    wid = lax.axis_index("s") * NC + lax.axis_index("c")
    lane = lax.broadcasted_iota(jnp.int32, (L,), 0)
    cks = [lane * H3 + ((lane + k) % 3) * NBINS for k in range(3)]
    ones = jnp.ones((L,), jnp.float32)
    zeros = jnp.zeros((L,), jnp.float32)

    for j in range(IPW):
        img = wid * IPW + j
        ibase = img * WPI

        def zero_body(i, c):
            hist[pl.ds(i * L, L)] = zeros
            return c
        lax.fori_loop(0, (L * H3) // L, zero_body, 0)

        pltpu.make_async_copy(
            in_hbm.at[pl.ds(ibase, CHUNK)], buf0, sem0).start()
        pltpu.make_async_copy(
            in_hbm.at[pl.ds(ibase + CHUNK, CHUNK)], buf1, sem1).start()

        for g in range(NCHUNK):
            buf, sem = (buf0, sem0) if g % 2 == 0 else (buf1, sem1)
            pltpu.make_async_copy(
                in_hbm.at[pl.ds(ibase + g * CHUNK, CHUNK)], buf, sem).wait()

            def chunk_body(t, c, buf=buf):
                for k in range(3):
                    v = buf[pl.ds(t * 48 + k * L, L)]
                    x = jnp.minimum(jnp.maximum(v * 256.0, 0.0), 255.0)
                    idx = x.astype(jnp.int32) + cks[k]
                    plsc.addupdate_scatter(hist, [idx], ones)
                return c
            lax.fori_loop(0, CHUNK // 48, chunk_body, 0)

            if g + 2 < NCHUNK:
                pltpu.make_async_copy(
                    in_hbm.at[pl.ds(ibase + (g + 2) * CHUNK, CHUNK)],
                    buf, sem).start()

        def red_body(v, c):
            acc = zeros
            for l in range(L):
                acc = acc + hist[pl.ds(l * H3 + v * L, L)]
            out_buf[pl.ds(v * L, L)] = acc / float(HW)
            return c
        lax.fori_loop(0, H3 // L, red_body, 0)

        pltpu.sync_copy(out_buf, out_hbm.at[img])


def _hist(flat):
    return pl.kernel(
        _body,
        out_type=jax.ShapeDtypeStruct((B, H3), jnp.float32),
        mesh=plsc.VectorSubcoreMesh(core_axis_name="c", subcore_axis_name="s"),
        compiler_params=pltpu.CompilerParams(needs_layout_passes=False),
        scratch_types=[
            pltpu.VMEM((CHUNK,), jnp.float32),
            pltpu.VMEM((CHUNK,), jnp.float32),
            pltpu.VMEM((L * H3,), jnp.float32),
            pltpu.VMEM((H3,), jnp.float32),
            pltpu.SemaphoreType.DMA,
            pltpu.SemaphoreType.DMA,
        ],
    )(flat)


def kernel(inputs):
    flat = inputs.reshape(-1)
    out = _hist(flat)
    return out.reshape(B, 3, NBINS).transpose(0, 2, 1)
